# Initial kernel scaffold; baseline (speedup 1.0000x reference)
import jax
import jax.numpy as jnp
from jax.experimental import pallas as pl


def kernel(batch, emb_weight):
    raise NotImplementedError("write your pallas kernel here")


# speedup vs baseline: 1.5593x; 1.5593x over previous
import functools

import jax
import jax.numpy as jnp
from jax import lax
from jax.experimental import pallas as pl
from jax.experimental.pallas import tpu as pltpu
from jax.experimental.pallas import tpu_sc as plsc

_N_DIM = 32
_NC = 2
_NS = 16
_NW = _NC * _NS


@functools.cache
def _build(B):
    bpw = B // _NW
    nch = 13
    ch = bpw // nch
    assert ch * nch == bpw and ch % 8 == 0

    mesh = plsc.VectorSubcoreMesh(core_axis_name="c", subcore_axis_name="s")

    @functools.partial(
        pl.kernel,
        mesh=mesh,
        out_type=jax.ShapeDtypeStruct((B, _N_DIM), jnp.float32),
        scratch_types=[
            pltpu.VMEM((bpw,), jnp.int32),
            pltpu.VMEM((2, ch, _N_DIM), jnp.float32),
            pltpu.SemaphoreType.DMA,
            pltpu.SemaphoreType.DMA,
        ],
        compiler_params=pltpu.CompilerParams(use_tc_tiling_on_sc=False),
    )
    def body(idx_hbm, table_hbm, out_hbm, idx_v, rows_v, gsem, ssem):
        wid = lax.axis_index("s") * _NC + lax.axis_index("c")
        base = wid * bpw
        pltpu.sync_copy(idx_hbm.at[pl.ds(base, bpw)], idx_v)
        for c in range(nch):
            buf = rows_v.at[c % 2]
            pltpu.async_copy(
                table_hbm.at[idx_v.at[pl.ds(c * ch, ch)]], buf, gsem
            ).wait()
            pltpu.async_copy(
                buf, out_hbm.at[pl.ds(base + c * ch, ch)], ssem
            ).wait()

    return body


def kernel(batch, emb_weight):
    b0, b1 = batch.shape
    idx = batch.reshape(-1).astype(jnp.int32)
    out = _build(idx.shape[0])(idx, emb_weight)
    return out.reshape(b0, b1, _N_DIM)


# speedup vs baseline: 1.5785x; 1.0123x over previous
import functools

import jax
import jax.numpy as jnp
from jax import lax
from jax.experimental import pallas as pl
from jax.experimental.pallas import tpu as pltpu
from jax.experimental.pallas import tpu_sc as plsc

_N_DIM = 32
_NC = 2
_NS = 16
_NW = _NC * _NS


@functools.cache
def _build(batch_n, fields):
    bpw = batch_n // _NW
    assert bpw * _NW == batch_n and (bpw * fields) % 8 == 0

    mesh = plsc.VectorSubcoreMesh(core_axis_name="c", subcore_axis_name="s")

    @functools.partial(
        pl.kernel,
        mesh=mesh,
        out_type=jax.ShapeDtypeStruct((batch_n, fields, _N_DIM), jnp.float32),
        scratch_types=[
            pltpu.VMEM((fields, bpw), jnp.int32),
            pltpu.VMEM((2, bpw, _N_DIM), jnp.float32),
            pltpu.SemaphoreType.DMA,
            pltpu.SemaphoreType.DMA,
        ],
        compiler_params=pltpu.CompilerParams(use_tc_tiling_on_sc=False),
    )
    def body(idxt_hbm, table_hbm, out_hbm, idx_v, rows_v, gsem, ssem):
        wid = lax.axis_index("s") * _NC + lax.axis_index("c")
        b0 = wid * bpw
        pltpu.sync_copy(idxt_hbm.at[:, pl.ds(b0, bpw)], idx_v)

        def gather(f):
            return pltpu.async_copy(
                table_hbm.at[idx_v.at[f]], rows_v.at[f % 2], gsem)

        def scatter(f):
            return pltpu.async_copy(
                rows_v.at[f % 2], out_hbm.at[pl.ds(b0, bpw), f], ssem)

        gathers = [gather(0)]
        scatters = []
        for f in range(fields):
            if f + 1 < fields:
                if f >= 1:
                    scatters[f - 1].wait()
                gathers.append(gather(f + 1))
            gathers[f].wait()
            scatters.append(scatter(f))
        scatters[fields - 2].wait()
        scatters[fields - 1].wait()

    return body


def kernel(batch, emb_weight):
    bn, fields = batch.shape
    n_emb, n_dim = emb_weight.shape
    idx_t = batch.astype(jnp.int32).T
    wide = jax.lax.optimization_barrier(
        emb_weight.reshape(n_emb * n_dim // 128, 128))
    table = wide.reshape(n_emb, n_dim)
    return _build(bn, fields)(idx_t, table)


# speedup vs baseline: 1.6723x; 1.0594x over previous
import functools

import jax
import jax.numpy as jnp
from jax import lax
from jax.experimental import pallas as pl
from jax.experimental.pallas import tpu as pltpu
from jax.experimental.pallas import tpu_sc as plsc

_N_DIM = 32
_PAD_W = 128
_NC = 2
_NS = 16
_NW = _NC * _NS


@functools.cache
def _build(batch_n, fields):
    bpw = batch_n // _NW
    assert bpw * _NW == batch_n

    mesh = plsc.VectorSubcoreMesh(core_axis_name="c", subcore_axis_name="s")

    @functools.partial(
        pl.kernel,
        mesh=mesh,
        out_type=jax.ShapeDtypeStruct((fields, batch_n, _PAD_W), jnp.float32),
        scratch_types=[
            pltpu.VMEM((fields, bpw), jnp.int32),
            pltpu.VMEM((2, bpw, _N_DIM), jnp.float32),
            pltpu.SemaphoreType.DMA,
            pltpu.SemaphoreType.DMA,
        ],
        compiler_params=pltpu.CompilerParams(use_tc_tiling_on_sc=False),
    )
    def body(idxt_hbm, table_hbm, out_hbm, idx_v, rows_v, gsem, ssem):
        wid = lax.axis_index("s") * _NC + lax.axis_index("c")
        b0 = wid * bpw
        pltpu.sync_copy(idxt_hbm.at[:, pl.ds(b0, bpw)], idx_v)

        def gather(f):
            return pltpu.async_copy(
                table_hbm.at[idx_v.at[f]], rows_v.at[f % 2], gsem)

        def scatter(f):
            return pltpu.async_copy(
                rows_v.at[f % 2],
                out_hbm.at[f, pl.ds(b0, bpw), pl.ds(0, _N_DIM)], ssem)

        gathers = [gather(0)]
        scatters = []
        for f in range(fields):
            if f + 1 < fields:
                if f >= 1:
                    scatters[f - 1].wait()
                gathers.append(gather(f + 1))
            gathers[f].wait()
            scatters.append(scatter(f))
        scatters[fields - 2].wait()
        scatters[fields - 1].wait()

    return body


def kernel(batch, emb_weight):
    bn, fields = batch.shape
    idx_t = batch.astype(jnp.int32).T
    out = _build(bn, fields)(idx_t, emb_weight)
    return out[:, :, :_N_DIM].transpose(1, 0, 2)


# speedup vs baseline: 2.0505x; 1.2261x over previous
import functools

import jax
import jax.numpy as jnp
from jax import lax
from jax.experimental import pallas as pl
from jax.experimental.pallas import tpu as pltpu
from jax.experimental.pallas import tpu_sc as plsc

_N_DIM = 32
_PAD_W = 128
_NC = 2
_NS = 16
_NW = _NC * _NS


@functools.cache
def _build(batch_n, fields):
    bpw = batch_n // _NW
    assert bpw * _NW == batch_n

    mesh = plsc.VectorSubcoreMesh(core_axis_name="c", subcore_axis_name="s")

    @functools.partial(
        pl.kernel,
        mesh=mesh,
        out_type=jax.ShapeDtypeStruct((fields, batch_n, _PAD_W), jnp.float32),
        scratch_types=[
            pltpu.VMEM((fields, bpw), jnp.int32),
            pltpu.VMEM((2, bpw, _N_DIM), jnp.float32),
            pltpu.SemaphoreType.DMA,
            pltpu.SemaphoreType.DMA,
        ],
        compiler_params=pltpu.CompilerParams(use_tc_tiling_on_sc=False),
    )
    def body(idxt_hbm, table_hbm, out_hbm, idx_v, rows_v, gsem, ssem):
        wid = lax.axis_index("s") * _NC + lax.axis_index("c")
        b0 = wid * bpw
        pltpu.sync_copy(idxt_hbm.at[:, pl.ds(b0, bpw)], idx_v)

        def gather(f):
            return pltpu.async_copy(
                table_hbm.at[idx_v.at[f]], rows_v.at[f % 2], gsem)

        def scatter(f):
            return pltpu.async_copy(
                rows_v.at[f % 2],
                out_hbm.at[f, pl.ds(b0, bpw), pl.ds(0, _N_DIM)], ssem)

        gathers = [gather(0)]
        scatters = []
        for f in range(fields):
            if f + 1 < fields:
                if f >= 1:
                    scatters[f - 1].wait()
                gathers.append(gather(f + 1))
            gathers[f].wait()
            scatters.append(scatter(f))
        scatters[fields - 2].wait()
        scatters[fields - 1].wait()

    return body


def kernel(batch, emb_weight):
    bn, fields = batch.shape
    idx_t = batch.astype(jnp.int32).T
    out = _build(bn, fields)(idx_t, emb_weight)
    return out.transpose(1, 0, 2)[:, :, :_N_DIM]
